# baseline (speedup 1.0000x reference)
import functools

import jax
import jax.numpy as jnp
import numpy as np
from jax import lax
from jax.experimental import pallas as pl
from jax.experimental.pallas import tpu as pltpu
from jax.experimental.pallas import tpu_sc as plsc

N = 10000
E = 160000
V = 100000
L = 50
D = 128
H = 8
HID = 64
C = 42
NEG = 0.2

NC = 2
NS = 16
NW = NC * NS
NPW = 320
NPAD = NW * NPW
KE = 128
KEA = 128
ECW = ((E + KE - 1) // KE) * KE + KE
ECAP = NW * ECW
CP = 48

_MESH = plsc.VectorSubcoreMesh(core_axis_name="c", subcore_axis_name="s")


def _wid():
    return lax.axis_index("s") * NC + lax.axis_index("c")


@functools.partial(
    pl.kernel,
    out_type=jax.ShapeDtypeStruct((NPAD, D), jnp.float32),
    mesh=_MESH,
    scratch_types=[
        pltpu.VMEM((NPW // 2, 104), jnp.int32),
        pltpu.VMEM((2, 104, D), jnp.float32),
        pltpu.VMEM((NPW, D), jnp.float32),
        pltpu.SemaphoreType.DMA,
    ],
)
def _encoder(inputs_hbm, emb_hbm, enc_hbm, idx_v, rows_v, out_v, sem):
    w = _wid()
    lo2 = w * (NPW // 2)
    pltpu.sync_copy(inputs_hbm.at[pl.ds(lo2, NPW // 2)], idx_v)
    pltpu.async_copy(emb_hbm.at[idx_v.at[0]], rows_v.at[0], sem)

    def step(i, b):
        nxt = i + 1

        @pl.when(nxt < NPW // 2)
        def _():
            pltpu.async_copy(emb_hbm.at[idx_v.at[nxt]], rows_v.at[1 - b], sem)

        pltpu.make_async_copy(emb_hbm.at[idx_v.at[i]], rows_v.at[b], sem).wait()
        for half in range(2):
            for c in range(D // 16):
                accs = [jnp.zeros((16,), jnp.float32) for _ in range(4)]
                for r in range(L):
                    accs[r % 4] = accs[r % 4] + rows_v[
                        b, half * L + r, pl.ds(c * 16, 16)]
                acc = (accs[0] + accs[1]) + (accs[2] + accs[3])
                out_v[2 * i + half, pl.ds(c * 16, 16)] = acc * (1.0 / L)

    def loop_body(j, carry):
        step(2 * j, 0)
        step(2 * j + 1, 1)
        return carry

    lax.fori_loop(0, NPW // 4, loop_body, 0)
    pltpu.sync_copy(out_v, enc_hbm.at[pl.ds(w * NPW, NPW)])


_CH = 8000


@functools.partial(
    pl.kernel,
    out_type=(
        jax.ShapeDtypeStruct((ECAP,), jnp.int32),
        jax.ShapeDtypeStruct((ECAP,), jnp.int32),
        jax.ShapeDtypeStruct((NW, 16), jnp.int32),
    ),
    mesh=_MESH,
    scratch_types=[
        pltpu.VMEM((_CH,), jnp.int32),
        pltpu.VMEM((_CH,), jnp.int32),
        pltpu.VMEM((KE + 80,), jnp.int32),
        pltpu.VMEM((KE + 80,), jnp.int32),
        pltpu.VMEM((16,), jnp.int32),
    ],
    compiler_params=pltpu.CompilerParams(needs_layout_passes=False),
)
def _fill(src_hbm, dst_hbm, srcs_hbm, dsts_hbm, cnt_hbm,
          srcb, dstb, stg_s, stg_d, cnt_v):
    w = _wid()
    lo = w * NPW
    hi = lo + NPW
    off = w * ECW

    for i in range((KE + 80) // 16):
        stg_s[pl.ds(i * 16, 16)] = jnp.zeros((16,), jnp.int32)
        stg_d[pl.ds(i * 16, 16)] = jnp.zeros((16,), jnp.int32)

    def outer(j, carry):
        pltpu.sync_copy(src_hbm.at[pl.ds(j * _CH, _CH)], srcb)
        pltpu.sync_copy(dst_hbm.at[pl.ds(j * _CH, _CH)], dstb)

        def inner(i, carry):
            p, base = carry

            @pl.when(p >= KE)
            def _():
                fo = pl.multiple_of(off + base, KE)
                pltpu.sync_copy(stg_s.at[pl.ds(0, KE)],
                                srcs_hbm.at[pl.ds(fo, KE)])
                pltpu.sync_copy(stg_d.at[pl.ds(0, KE)],
                                dsts_hbm.at[pl.ds(fo, KE)])
                for q in range(5):
                    sv = stg_s[pl.ds(KE + q * 16, 16)]
                    dv = stg_d[pl.ds(KE + q * 16, 16)]
                    stg_s[pl.ds(q * 16, 16)] = sv
                    stg_d[pl.ds(q * 16, 16)] = dv

            base = jnp.where(p >= KE, base + KE, base)
            p = jnp.where(p >= KE, p - KE, p)

            ila = lax.iota(jnp.int32, 16)
            pcs = []
            perms = []
            svs = []
            dvs = []
            for g in range(4):
                d = dstb[pl.ds(i * 64 + g * 16, 16)]
                s = srcb[pl.ds(i * 64 + g * 16, 16)]
                m = (d >= lo) & (d < hi)
                k = jnp.where(m, ila, ila + 16)
                _, perm = plsc.sort_key_val(k, ila)
                svs.append(jnp.take(s, perm))
                dvs.append(jnp.take(d, perm))
                perms.append(perm)
                pcs.append(plsc.all_reduce_population_count(m)[0])
            for g in range(4):
                stg_s[pl.ds(p, 16)] = svs[g]
                stg_d[pl.ds(p, 16)] = dvs[g]
                p = p + pcs[g]
            return (p, base)

        return lax.fori_loop(0, _CH // 64, inner, carry)

    p, base = lax.fori_loop(0, E // _CH, outer,
                            (jnp.int32(0), jnp.int32(0)))
    fo = pl.multiple_of(off + base, KE)
    pltpu.sync_copy(stg_s.at[pl.ds(0, KE)], srcs_hbm.at[pl.ds(fo, KE)])
    pltpu.sync_copy(stg_d.at[pl.ds(0, KE)], dsts_hbm.at[pl.ds(fo, KE)])

    @pl.when(p > KE)
    def _():
        fo2 = pl.multiple_of(off + base + KE, KE)
        for q in range(5):
            stg_s[pl.ds(q * 16, 16)] = stg_s[pl.ds(KE + q * 16, 16)]
            stg_d[pl.ds(q * 16, 16)] = stg_d[pl.ds(KE + q * 16, 16)]
        pltpu.sync_copy(stg_s.at[pl.ds(0, KE)], srcs_hbm.at[pl.ds(fo2, KE)])
        pltpu.sync_copy(stg_d.at[pl.ds(0, KE)], dsts_hbm.at[pl.ds(fo2, KE)])

    cnt_v[...] = jnp.zeros((16,), jnp.int32) + (base + p)
    pltpu.sync_copy(cnt_v, cnt_hbm.at[w])


@functools.partial(
    pl.kernel,
    out_type=(
        jax.ShapeDtypeStruct((ECAP, 16), jnp.float32),
        jax.ShapeDtypeStruct((NPAD, 16), jnp.float32),
    ),
    mesh=_MESH,
    scratch_types=[
        pltpu.VMEM((KEA + 16,), jnp.int32),
        pltpu.VMEM((KEA + 16,), jnp.int32),
        pltpu.VMEM((KEA, 128), jnp.float32),
        pltpu.VMEM((KEA, 128), jnp.float32),
        pltpu.VMEM((KEA, 16), jnp.float32),
        pltpu.VMEM((NPW, 16), jnp.float32),
        pltpu.VMEM((16,), jnp.int32),
        pltpu.SemaphoreType.DMA,
    ],
)
def _att(s_hbm, d_hbm, srcs_hbm, dsts_hbm, cnt_hbm, att_hbm, den_hbm,
         srcb, dstb, sS, sD, attb, den_v, meta_v, sem):
    w = _wid()
    lo = w * NPW
    pltpu.sync_copy(cnt_hbm.at[w], meta_v)
    off = w * ECW
    cnt = meta_v[...][0]

    def zinit(i, c):
        den_v[i] = jnp.zeros((16,), jnp.float32)
        return c

    lax.fori_loop(0, NPW, zinit, 0)

    nchunks = (cnt + (KEA - 1)) // KEA

    def chunk(j, c):
        base = pl.multiple_of(off + j * KEA, KEA)
        pltpu.sync_copy(srcs_hbm.at[pl.ds(base, KEA)], srcb.at[pl.ds(0, KEA)])
        pltpu.sync_copy(dsts_hbm.at[pl.ds(base, KEA)], dstb.at[pl.ds(0, KEA)])
        si = srcb.at[pl.ds(0, KEA)]
        di = dstb.at[pl.ds(0, KEA)]
        pltpu.async_copy(s_hbm.at[si], sS, sem)
        pltpu.async_copy(d_hbm.at[di], sD, sem)
        pltpu.make_async_copy(s_hbm.at[si], sS, sem).wait()
        pltpu.make_async_copy(d_hbm.at[di], sD, sem).wait()
        ne = jnp.minimum(KEA, cnt - j * KEA)

        @functools.partial(plsc.parallel_loop, 0, ne, unroll=4)
        def edge(e):
            x = sS[e, pl.ds(0, 16)] + sD[e, pl.ds(0, 16)]
            e8 = jnp.where(x >= 0, x, NEG * x)
            att8 = jnp.exp(e8)
            attb[e] = att8
            dl = dstb[pl.ds(e, 16)][0] - lo
            plsc.addupdate(den_v.at[dl], att8)

        pltpu.sync_copy(attb, att_hbm.at[pl.ds(base, KEA)])
        return c

    lax.fori_loop(0, nchunks, chunk, 0)
    pltpu.sync_copy(den_v, den_hbm.at[pl.ds(lo, NPW)])


def _make_agg(cg, cgpad, nheads, lane0):
    chw = cg // nheads

    @functools.partial(
        pl.kernel,
        out_type=jax.ShapeDtypeStruct((NPAD, cg), jnp.float32),
        mesh=_MESH,
        scratch_types=[
            pltpu.VMEM((KE + 16,), jnp.int32),
            pltpu.VMEM((KE + 16,), jnp.int32),
            pltpu.VMEM((KE, 16), jnp.float32),
            pltpu.VMEM((KE, cgpad), jnp.float32),
            pltpu.VMEM((NPW, cg), jnp.float32),
            pltpu.VMEM((NPW, 16), jnp.float32),
            pltpu.VMEM((16,), jnp.int32),
        pltpu.SemaphoreType.DMA,
        ],
    )
    def _agg(w_hbm, att_hbm, srcs_hbm, dsts_hbm, cnt_hbm, den_hbm, x_hbm,
             srcb, dstb, attb, rowb, acc_v, den_v, meta_v, sem):
        w = _wid()
        lo = w * NPW
        pltpu.sync_copy(cnt_hbm.at[w], meta_v)
        off = w * ECW
        cnt = meta_v[...][0]

        def zinit(i, c):
            for cc in range(cg // 16):
                acc_v[i, pl.ds(cc * 16, 16)] = jnp.zeros((16,), jnp.float32)
            return c

        lax.fori_loop(0, NPW, zinit, 0)

        nchunks = (cnt + (KE - 1)) // KE

        def chunk(j, c):
            base = pl.multiple_of(off + j * KE, KE)
            pltpu.sync_copy(srcs_hbm.at[pl.ds(base, KE)], srcb.at[pl.ds(0, KE)])
            pltpu.sync_copy(dsts_hbm.at[pl.ds(base, KE)], dstb.at[pl.ds(0, KE)])
            pltpu.sync_copy(att_hbm.at[pl.ds(base, KE)], attb)
            si = srcb.at[pl.ds(0, KE)]
            pltpu.async_copy(w_hbm.at[si], rowb, sem)
            pltpu.make_async_copy(w_hbm.at[si], rowb, sem).wait()
            ne = jnp.minimum(KE, cnt - j * KE)

            @functools.partial(plsc.parallel_loop, 0, ne, unroll=4)
            def edge(e):
                dl = dstb[pl.ds(e, 16)][0] - lo
                av = attb[e]
                for h in range(nheads):
                    a_vec = av[lane0 + h]
                    for cc in range(chw // 16):
                        col = h * chw + cc * 16
                        plsc.addupdate(acc_v.at[dl, pl.ds(col, 16)],
                                       a_vec * rowb[e, pl.ds(col, 16)])
            return c

        lax.fori_loop(0, nchunks, chunk, 0)

        pltpu.sync_copy(den_hbm.at[pl.ds(lo, NPW)], den_v)

        def norm(i, c):
            dnv = den_v[i]
            for h in range(nheads):
                dh = jnp.zeros((16,), jnp.float32) + dnv[lane0 + h]
                sc = 1.0 / (dh + 1e-10)
                for cc in range(chw // 16):
                    col = h * chw + cc * 16
                    x = acc_v[i, pl.ds(col, 16)] * sc
                    acc_v[i, pl.ds(col, 16)] = jnp.where(
                        x >= 0, x, jnp.exp(x) - 1.0)
            return c

        lax.fori_loop(0, NPW, norm, 0)
        pltpu.sync_copy(acc_v, x_hbm.at[pl.ds(lo, NPW)])

    return _agg


_agg_l1 = [_make_agg(2 * HID, 2 * HID, 2, 2 * g) for g in range(4)]
_agg_l2 = _make_agg(CP, 128, 1, 0)


_LSM_BLK = 512


def _lsm_body(x_ref, o_ref):
    x = x_ref[...]
    m = jnp.max(x, axis=1, keepdims=True)
    ex = jnp.exp(x - m)
    o_ref[...] = (x - m) - jnp.log(jnp.sum(ex, axis=1, keepdims=True))


def _log_softmax(x):
    return pl.pallas_call(
        _lsm_body,
        out_shape=jax.ShapeDtypeStruct(x.shape, x.dtype),
        grid=(x.shape[0] // _LSM_BLK,),
        in_specs=[pl.BlockSpec((_LSM_BLK, x.shape[1]), lambda i: (i, 0))],
        out_specs=pl.BlockSpec((_LSM_BLK, x.shape[1]), lambda i: (i, 0)),
    )(x)


def kernel(inputs, adj, emb, W1, a_src, a_dst, W2, ao_src, ao_dst):
    src = adj[0]
    dst = adj[1]
    inputs_pad = jnp.zeros((NPAD, L), jnp.int32).at[:N].set(inputs)
    idx_pairs = jnp.zeros((NPAD // 2, 104), jnp.int32)
    idx_pairs = idx_pairs.at[:, :2 * L].set(inputs_pad.reshape(NPAD // 2, 2 * L))
    enc_pad = _encoder(idx_pairs, emb)
    enc = enc_pad[:N]

    srcs, dsts, cnts = _fill(src, dst)

    W1cat = jnp.transpose(W1, (1, 0, 2)).reshape(D, H * HID)
    Wh = enc_pad @ W1cat
    As = jnp.einsum("hdk,hk->dh", W1, a_src)
    Ad = jnp.einsum("hdk,hk->dh", W1, a_dst)
    S1 = jnp.pad(enc_pad @ As, ((0, 0), (0, 128 - H)))
    D1 = jnp.pad(enc_pad @ Ad, ((0, 0), (0, 128 - H)))

    att1, den1 = _att(S1, D1, srcs, dsts, cnts)
    x1_parts = [
        _agg_l1[g](Wh[:, g * 2 * HID:(g + 1) * 2 * HID],
                   att1, srcs, dsts, cnts, den1)
        for g in range(4)
    ]
    x1 = jnp.concatenate(x1_parts, axis=1)

    Wh2 = jnp.pad(x1 @ W2, ((0, 0), (0, 128 - C)))
    s2 = x1 @ (W2 @ ao_src)
    d2 = x1 @ (W2 @ ao_dst)
    S2 = jnp.zeros((NPAD, 128), jnp.float32).at[:, 0].set(s2)
    D2 = jnp.zeros((NPAD, 128), jnp.float32).at[:, 0].set(d2)

    att2, den2 = _att(S2, D2, srcs, dsts, cnts)
    h2 = _agg_l2(Wh2, att2, srcs, dsts, cnts, den2)

    logits = _log_softmax(h2[:, :C])[:N]
    return (logits, enc)


# speedup vs baseline: 1.3833x; 1.3833x over previous
import functools

import jax
import jax.numpy as jnp
import numpy as np
from jax import lax
from jax.experimental import pallas as pl
from jax.experimental.pallas import tpu as pltpu
from jax.experimental.pallas import tpu_sc as plsc

N = 10000
E = 160000
V = 100000
L = 50
D = 128
H = 8
HID = 64
C = 42
NEG = 0.2

NC = 2
NS = 16
NW = NC * NS
NPW = 320
NPAD = NW * NPW
KE = 128
KEA = 128
ECW = ((E + KE - 1) // KE) * KE + KE
ECAP = NW * ECW
CP = 48

_MESH = plsc.VectorSubcoreMesh(core_axis_name="c", subcore_axis_name="s")


def _wid():
    return lax.axis_index("s") * NC + lax.axis_index("c")


@functools.partial(
    pl.kernel,
    out_type=jax.ShapeDtypeStruct((NPAD, D), jnp.float32),
    mesh=_MESH,
    scratch_types=[
        pltpu.VMEM((NPW, L), jnp.int32),
        pltpu.VMEM((4, L, D), jnp.float32),
        pltpu.VMEM((NPW, D), jnp.float32),
        pltpu.SemaphoreType.DMA,
    ],
)
def _encoder(inputs_hbm, emb_hbm, enc_hbm, idx_v, rows_v, out_v, sem):
    w = _wid()
    lo = w * NPW
    pltpu.sync_copy(inputs_hbm.at[pl.ds(lo, NPW)], idx_v)
    for b in range(3):
        pltpu.async_copy(emb_hbm.at[idx_v.at[b]], rows_v.at[b], sem)

    def step(i, b):
        nxt = i + 3

        @pl.when(nxt < NPW)
        def _():
            pltpu.async_copy(emb_hbm.at[idx_v.at[nxt]], rows_v.at[(b + 3) % 4],
                             sem)

        pltpu.make_async_copy(emb_hbm.at[idx_v.at[i]], rows_v.at[b], sem).wait()
        for c in range(D // 16):
            accs = [jnp.zeros((16,), jnp.float32) for _ in range(4)]
            for r in range(L):
                accs[r % 4] = accs[r % 4] + rows_v[b, r, pl.ds(c * 16, 16)]
            acc = (accs[0] + accs[1]) + (accs[2] + accs[3])
            out_v[i, pl.ds(c * 16, 16)] = acc * (1.0 / L)

    def loop_body(j, carry):
        for k in range(4):
            step(4 * j + k, k)
        return carry

    lax.fori_loop(0, NPW // 4, loop_body, 0)
    pltpu.sync_copy(out_v, enc_hbm.at[pl.ds(lo, NPW)])


_CH = 8000


@functools.partial(
    pl.kernel,
    out_type=(
        jax.ShapeDtypeStruct((ECAP,), jnp.int32),
        jax.ShapeDtypeStruct((ECAP,), jnp.int32),
        jax.ShapeDtypeStruct((NW, 16), jnp.int32),
    ),
    mesh=_MESH,
    scratch_types=[
        pltpu.VMEM((_CH,), jnp.int32),
        pltpu.VMEM((_CH,), jnp.int32),
        pltpu.VMEM((KE + 80,), jnp.int32),
        pltpu.VMEM((KE + 80,), jnp.int32),
        pltpu.VMEM((16,), jnp.int32),
    ],
    compiler_params=pltpu.CompilerParams(needs_layout_passes=False),
)
def _fill(src_hbm, dst_hbm, srcs_hbm, dsts_hbm, cnt_hbm,
          srcb, dstb, stg_s, stg_d, cnt_v):
    w = _wid()
    lo = w * NPW
    hi = lo + NPW
    off = w * ECW

    for i in range((KE + 80) // 16):
        stg_s[pl.ds(i * 16, 16)] = jnp.zeros((16,), jnp.int32)
        stg_d[pl.ds(i * 16, 16)] = jnp.zeros((16,), jnp.int32)

    def outer(j, carry):
        pltpu.sync_copy(src_hbm.at[pl.ds(j * _CH, _CH)], srcb)
        pltpu.sync_copy(dst_hbm.at[pl.ds(j * _CH, _CH)], dstb)

        def inner(i, carry):
            p, base = carry

            @pl.when(p >= KE)
            def _():
                fo = pl.multiple_of(off + base, KE)
                pltpu.sync_copy(stg_s.at[pl.ds(0, KE)],
                                srcs_hbm.at[pl.ds(fo, KE)])
                pltpu.sync_copy(stg_d.at[pl.ds(0, KE)],
                                dsts_hbm.at[pl.ds(fo, KE)])
                for q in range(5):
                    sv = stg_s[pl.ds(KE + q * 16, 16)]
                    dv = stg_d[pl.ds(KE + q * 16, 16)]
                    stg_s[pl.ds(q * 16, 16)] = sv
                    stg_d[pl.ds(q * 16, 16)] = dv

            base = jnp.where(p >= KE, base + KE, base)
            p = jnp.where(p >= KE, p - KE, p)

            ila = lax.iota(jnp.int32, 16)
            pcs = []
            perms = []
            svs = []
            dvs = []
            for g in range(4):
                d = dstb[pl.ds(i * 64 + g * 16, 16)]
                s = srcb[pl.ds(i * 64 + g * 16, 16)]
                m = (d >= lo) & (d < hi)
                k = jnp.where(m, ila, ila + 16)
                _, perm = plsc.sort_key_val(k, ila)
                svs.append(jnp.take(s, perm))
                dvs.append(jnp.take(d, perm))
                perms.append(perm)
                pcs.append(plsc.all_reduce_population_count(m)[0])
            for g in range(4):
                stg_s[pl.ds(p, 16)] = svs[g]
                stg_d[pl.ds(p, 16)] = dvs[g]
                p = p + pcs[g]
            return (p, base)

        return lax.fori_loop(0, _CH // 64, inner, carry)

    p, base = lax.fori_loop(0, E // _CH, outer,
                            (jnp.int32(0), jnp.int32(0)))
    fo = pl.multiple_of(off + base, KE)
    pltpu.sync_copy(stg_s.at[pl.ds(0, KE)], srcs_hbm.at[pl.ds(fo, KE)])
    pltpu.sync_copy(stg_d.at[pl.ds(0, KE)], dsts_hbm.at[pl.ds(fo, KE)])

    @pl.when(p > KE)
    def _():
        fo2 = pl.multiple_of(off + base + KE, KE)
        for q in range(5):
            stg_s[pl.ds(q * 16, 16)] = stg_s[pl.ds(KE + q * 16, 16)]
            stg_d[pl.ds(q * 16, 16)] = stg_d[pl.ds(KE + q * 16, 16)]
        pltpu.sync_copy(stg_s.at[pl.ds(0, KE)], srcs_hbm.at[pl.ds(fo2, KE)])
        pltpu.sync_copy(stg_d.at[pl.ds(0, KE)], dsts_hbm.at[pl.ds(fo2, KE)])

    cnt_v[...] = jnp.zeros((16,), jnp.int32) + (base + p)
    pltpu.sync_copy(cnt_v, cnt_hbm.at[w])


@functools.partial(
    pl.kernel,
    out_type=(
        jax.ShapeDtypeStruct((ECAP, 16), jnp.float32),
        jax.ShapeDtypeStruct((NPAD, 16), jnp.float32),
    ),
    mesh=_MESH,
    scratch_types=[
        pltpu.VMEM((KEA + 16,), jnp.int32),
        pltpu.VMEM((KEA + 16,), jnp.int32),
        pltpu.VMEM((KEA, 128), jnp.float32),
        pltpu.VMEM((KEA, 128), jnp.float32),
        pltpu.VMEM((KEA, 16), jnp.float32),
        pltpu.VMEM((NPW, 16), jnp.float32),
        pltpu.VMEM((16,), jnp.int32),
        pltpu.SemaphoreType.DMA,
    ],
)
def _att(s_hbm, d_hbm, srcs_hbm, dsts_hbm, cnt_hbm, att_hbm, den_hbm,
         srcb, dstb, sS, sD, attb, den_v, meta_v, sem):
    w = _wid()
    lo = w * NPW
    pltpu.sync_copy(cnt_hbm.at[w], meta_v)
    off = w * ECW
    cnt = meta_v[...][0]

    def zinit(i, c):
        den_v[i] = jnp.zeros((16,), jnp.float32)
        return c

    lax.fori_loop(0, NPW, zinit, 0)

    nchunks = (cnt + (KEA - 1)) // KEA

    def chunk(j, c):
        base = pl.multiple_of(off + j * KEA, KEA)
        pltpu.sync_copy(srcs_hbm.at[pl.ds(base, KEA)], srcb.at[pl.ds(0, KEA)])
        pltpu.sync_copy(dsts_hbm.at[pl.ds(base, KEA)], dstb.at[pl.ds(0, KEA)])
        si = srcb.at[pl.ds(0, KEA)]
        di = dstb.at[pl.ds(0, KEA)]
        pltpu.async_copy(s_hbm.at[si], sS, sem)
        pltpu.async_copy(d_hbm.at[di], sD, sem)
        pltpu.make_async_copy(s_hbm.at[si], sS, sem).wait()
        pltpu.make_async_copy(d_hbm.at[di], sD, sem).wait()
        ne = jnp.minimum(KEA, cnt - j * KEA)

        @functools.partial(plsc.parallel_loop, 0, ne, unroll=4)
        def edge(e):
            x = sS[e, pl.ds(0, 16)] + sD[e, pl.ds(0, 16)]
            e8 = jnp.where(x >= 0, x, NEG * x)
            att8 = jnp.exp(e8)
            attb[e] = att8
            dl = dstb[pl.ds(e, 16)][0] - lo
            plsc.addupdate(den_v.at[dl], att8)

        pltpu.sync_copy(attb, att_hbm.at[pl.ds(base, KEA)])
        return c

    lax.fori_loop(0, nchunks, chunk, 0)
    pltpu.sync_copy(den_v, den_hbm.at[pl.ds(lo, NPW)])


def _make_agg(cg, cgpad, nheads, lane0):
    chw = cg // nheads

    @functools.partial(
        pl.kernel,
        out_type=jax.ShapeDtypeStruct((NPAD, cg), jnp.float32),
        mesh=_MESH,
        scratch_types=[
            pltpu.VMEM((KE + 16,), jnp.int32),
            pltpu.VMEM((KE + 16,), jnp.int32),
            pltpu.VMEM((KE, 16), jnp.float32),
            pltpu.VMEM((KE, cgpad), jnp.float32),
            pltpu.VMEM((NPW, cg), jnp.float32),
            pltpu.VMEM((NPW, 16), jnp.float32),
            pltpu.VMEM((16,), jnp.int32),
        pltpu.SemaphoreType.DMA,
        ],
    )
    def _agg(w_hbm, att_hbm, srcs_hbm, dsts_hbm, cnt_hbm, den_hbm, x_hbm,
             srcb, dstb, attb, rowb, acc_v, den_v, meta_v, sem):
        w = _wid()
        lo = w * NPW
        pltpu.sync_copy(cnt_hbm.at[w], meta_v)
        off = w * ECW
        cnt = meta_v[...][0]

        def zinit(i, c):
            for cc in range(cg // 16):
                acc_v[i, pl.ds(cc * 16, 16)] = jnp.zeros((16,), jnp.float32)
            return c

        lax.fori_loop(0, NPW, zinit, 0)

        nchunks = (cnt + (KE - 1)) // KE

        def chunk(j, c):
            base = pl.multiple_of(off + j * KE, KE)
            pltpu.sync_copy(srcs_hbm.at[pl.ds(base, KE)], srcb.at[pl.ds(0, KE)])
            pltpu.sync_copy(dsts_hbm.at[pl.ds(base, KE)], dstb.at[pl.ds(0, KE)])
            pltpu.sync_copy(att_hbm.at[pl.ds(base, KE)], attb)
            si = srcb.at[pl.ds(0, KE)]
            pltpu.async_copy(w_hbm.at[si], rowb, sem)
            pltpu.make_async_copy(w_hbm.at[si], rowb, sem).wait()
            ne = jnp.minimum(KE, cnt - j * KE)

            @functools.partial(plsc.parallel_loop, 0, ne, unroll=4)
            def edge(e):
                dl = dstb[pl.ds(e, 16)][0] - lo
                av = attb[e]
                for h in range(nheads):
                    a_vec = av[lane0 + h]
                    for cc in range(chw // 16):
                        col = h * chw + cc * 16
                        plsc.addupdate(acc_v.at[dl, pl.ds(col, 16)],
                                       a_vec * rowb[e, pl.ds(col, 16)])
            return c

        lax.fori_loop(0, nchunks, chunk, 0)

        pltpu.sync_copy(den_hbm.at[pl.ds(lo, NPW)], den_v)

        def norm(i, c):
            dnv = den_v[i]
            for h in range(nheads):
                dh = jnp.zeros((16,), jnp.float32) + dnv[lane0 + h]
                sc = 1.0 / (dh + 1e-10)
                for cc in range(chw // 16):
                    col = h * chw + cc * 16
                    x = acc_v[i, pl.ds(col, 16)] * sc
                    acc_v[i, pl.ds(col, 16)] = jnp.where(
                        x >= 0, x, jnp.exp(x) - 1.0)
            return c

        lax.fori_loop(0, NPW, norm, 0)
        pltpu.sync_copy(acc_v, x_hbm.at[pl.ds(lo, NPW)])

    return _agg


_agg_l1 = [_make_agg(2 * HID, 2 * HID, 2, 2 * g) for g in range(4)]
_agg_l2 = _make_agg(CP, 128, 1, 0)


_LSM_BLK = 512


def _lsm_body(x_ref, o_ref):
    x = x_ref[...]
    m = jnp.max(x, axis=1, keepdims=True)
    ex = jnp.exp(x - m)
    o_ref[...] = (x - m) - jnp.log(jnp.sum(ex, axis=1, keepdims=True))


def _log_softmax(x):
    return pl.pallas_call(
        _lsm_body,
        out_shape=jax.ShapeDtypeStruct(x.shape, x.dtype),
        grid=(x.shape[0] // _LSM_BLK,),
        in_specs=[pl.BlockSpec((_LSM_BLK, x.shape[1]), lambda i: (i, 0))],
        out_specs=pl.BlockSpec((_LSM_BLK, x.shape[1]), lambda i: (i, 0)),
    )(x)


def kernel(inputs, adj, emb, W1, a_src, a_dst, W2, ao_src, ao_dst):
    src = adj[0]
    dst = adj[1]
    inputs_pad = jnp.zeros((NPAD, L), jnp.int32).at[:N].set(inputs)
    enc_pad = _encoder(inputs_pad, emb)
    enc = enc_pad[:N]

    srcs, dsts, cnts = _fill(src, dst)

    W1cat = jnp.transpose(W1, (1, 0, 2)).reshape(D, H * HID)
    Wh = enc_pad @ W1cat
    As = jnp.einsum("hdk,hk->dh", W1, a_src)
    Ad = jnp.einsum("hdk,hk->dh", W1, a_dst)
    S1 = jnp.pad(enc_pad @ As, ((0, 0), (0, 128 - H)))
    D1 = jnp.pad(enc_pad @ Ad, ((0, 0), (0, 128 - H)))

    att1, den1 = _att(S1, D1, srcs, dsts, cnts)
    x1_parts = [
        _agg_l1[g](Wh[:, g * 2 * HID:(g + 1) * 2 * HID],
                   att1, srcs, dsts, cnts, den1)
        for g in range(4)
    ]
    x1 = jnp.concatenate(x1_parts, axis=1)

    Wh2 = jnp.pad(x1 @ W2, ((0, 0), (0, 128 - C)))
    s2 = x1 @ (W2 @ ao_src)
    d2 = x1 @ (W2 @ ao_dst)
    S2 = jnp.zeros((NPAD, 128), jnp.float32).at[:, 0].set(s2)
    D2 = jnp.zeros((NPAD, 128), jnp.float32).at[:, 0].set(d2)

    att2, den2 = _att(S2, D2, srcs, dsts, cnts)
    h2 = _agg_l2(Wh2, att2, srcs, dsts, cnts, den2)

    logits = _log_softmax(h2[:, :C])[:N]
    return (logits, enc)
